# baseline (speedup 1.0000x reference)
import jax
import jax.numpy as jnp
from jax.experimental import pallas as pl
from jax.experimental.pallas import tpu as pltpu

VOCAB = 32768
HIDDEN = 2048
NUM_EXPERTS = 16
TOKENS = 512
EXPERT_VOCAB = VOCAB // NUM_EXPERTS
VSPLIT = 2
EVBLK = EXPERT_VOCAB // VSPLIT


def _moe_head_body(x_ref, w_ref, rw_ref, out_ref):
    e = pl.program_id(0)
    v = pl.program_id(1)
    x = x_ref[...]
    logits = jnp.dot(x, rw_ref[...].T, preferred_element_type=jnp.float32)
    m = jnp.max(logits, axis=1, keepdims=True)
    w = jnp.exp(logits - m)
    w = w / jnp.sum(w, axis=1, keepdims=True)
    a1 = jnp.argmax(w, axis=1)
    eids = jax.lax.broadcasted_iota(jnp.int32, (TOKENS, NUM_EXPERTS), 1)
    w2 = jnp.where(eids == a1[:, None], -jnp.inf, w)
    a2 = jnp.argmax(w2, axis=1)
    selcol = ((a1 == e) | (a2 == e))[:, None]

    xb = x.astype(jnp.bfloat16)
    wb = w_ref[0].astype(jnp.bfloat16)
    acc = jnp.dot(xb, wb.T, preferred_element_type=jnp.float32)
    out_ref[...] = jnp.where(selcol, acc, -jnp.inf)


def kernel(hidden_states, expert_weight, router_weight):
    return pl.pallas_call(
        _moe_head_body,
        grid=(NUM_EXPERTS, VSPLIT),
        in_specs=[
            pl.BlockSpec((TOKENS, HIDDEN), lambda e, v: (0, 0)),
            pl.BlockSpec((1, EVBLK, HIDDEN), lambda e, v: (e, v, 0)),
            pl.BlockSpec((NUM_EXPERTS, HIDDEN), lambda e, v: (0, 0)),
        ],
        out_specs=pl.BlockSpec((TOKENS, EVBLK),
                               lambda e, v: (0, e * VSPLIT + v)),
        out_shape=jax.ShapeDtypeStruct((TOKENS, VOCAB), jnp.float32),
        compiler_params=pltpu.CompilerParams(
            dimension_semantics=("parallel", "arbitrary")),
    )(hidden_states, expert_weight, router_weight)


# speedup vs baseline: 1.0878x; 1.0878x over previous
import jax
import jax.numpy as jnp
from jax.experimental import pallas as pl
from jax.experimental.pallas import tpu as pltpu

VOCAB = 32768
HIDDEN = 2048
NUM_EXPERTS = 16
TOKENS = 512
EXPERT_VOCAB = VOCAB // NUM_EXPERTS


def _moe_head_body(x_ref, w_ref, rw_ref, out_ref):
    e = pl.program_id(0)
    x = x_ref[...]
    logits = jnp.dot(x, rw_ref[...].T, preferred_element_type=jnp.float32)
    m = jnp.max(logits, axis=1, keepdims=True)
    w = jnp.exp(logits - m)
    w = w / jnp.sum(w, axis=1, keepdims=True)
    a1 = jnp.argmax(w, axis=1)
    eids = jax.lax.broadcasted_iota(jnp.int32, (TOKENS, NUM_EXPERTS), 1)
    w2 = jnp.where(eids == a1[:, None], -jnp.inf, w)
    a2 = jnp.argmax(w2, axis=1)
    selcol = ((a1 == e) | (a2 == e))[:, None]

    xb = x.astype(jnp.bfloat16)
    wb = w_ref[0].astype(jnp.bfloat16)
    acc = jnp.dot(xb, wb.T, preferred_element_type=jnp.float32)
    out_ref[...] = jnp.where(selcol, acc, -jnp.inf)


def kernel(hidden_states, expert_weight, router_weight):
    return pl.pallas_call(
        _moe_head_body,
        grid=(NUM_EXPERTS,),
        in_specs=[
            pl.BlockSpec((TOKENS, HIDDEN), lambda e: (0, 0)),
            pl.BlockSpec((1, EXPERT_VOCAB, HIDDEN), lambda e: (e, 0, 0)),
            pl.BlockSpec((NUM_EXPERTS, HIDDEN), lambda e: (0, 0)),
        ],
        out_specs=pl.BlockSpec((TOKENS, EXPERT_VOCAB), lambda e: (0, e)),
        out_shape=jax.ShapeDtypeStruct((TOKENS, VOCAB), jnp.float32),
        compiler_params=pltpu.CompilerParams(
            dimension_semantics=("parallel",)),
    )(hidden_states, expert_weight, router_weight)
